# Initial kernel scaffold; baseline (speedup 1.0000x reference)
import jax
import jax.numpy as jnp
from jax.experimental import pallas as pl


def kernel(x, W_think, b_think, idx, W_q, b_q):
    raise NotImplementedError("write your pallas kernel here")


# speedup vs baseline: 187.6130x; 187.6130x over previous
import functools

import jax
import jax.numpy as jnp
from jax import lax
from jax.experimental import pallas as pl
from jax.experimental.pallas import tpu as pltpu
from jax.experimental.pallas import tpu_sc as plsc


def _sc_motor_kernel(n_motor, n_conn, batch, n_neurons):
    info = plsc.get_sparse_core_info()
    nc, ns = info.num_cores, info.num_subcores
    nw = nc * ns
    npw = n_motor // nw
    rpw = npw * n_conn
    n_chunks = rpw // 128
    assert n_motor % nw == 0 and rpw % 128 == 0

    mesh = plsc.VectorSubcoreMesh(core_axis_name="c", subcore_axis_name="s")

    @functools.partial(
        pl.kernel,
        out_type=jax.ShapeDtypeStruct((n_motor, batch), jnp.float32),
        mesh=mesh,
        compiler_params=pltpu.CompilerParams(use_tc_tiling_on_sc=False),
        scratch_types=[
            pltpu.VMEM((n_chunks, 128), jnp.int32),
            pltpu.VMEM((rpw, batch), jnp.float32),
            pltpu.VMEM((rpw, batch), jnp.float32),
            pltpu.VMEM((npw, batch), jnp.float32),
            pltpu.VMEM((npw, batch), jnp.float32),
            pltpu.SemaphoreType.DMA,
        ],
    )
    def k(idx_hbm, wb_hbm, bb_hbm, xt_hbm, out_hbm,
          idx_v, rows_v, w_v, b_v, out_v, sem):
        wid = lax.axis_index("s") * nc + lax.axis_index("c")
        pltpu.sync_copy(idx_hbm.at[wid], idx_v)
        pltpu.sync_copy(wb_hbm.at[wid], w_v)
        pltpu.sync_copy(bb_hbm.at[wid], b_v)
        cps = [
            pltpu.async_copy(
                xt_hbm.at[idx_v.at[c]], rows_v.at[pl.ds(c * 128, 128)], sem)
            for c in range(n_chunks)
        ]
        for cp in cps:
            cp.wait()
        for n in range(npw):
            acc = b_v[n, :]
            for c in range(n_conn):
                r = n * n_conn + c
                acc = acc + rows_v[r, :] * w_v[r, :]
            out_v[n, :] = 1.0 / (1.0 + jnp.exp(-acc))
        pltpu.sync_copy(out_v, out_hbm.at[pl.ds(wid * npw, npw)])

    return k


def _q_head(m_ref, wq_ref, bq_ref, o_ref):
    q = lax.dot_general(
        m_ref[...], wq_ref[...],
        dimension_numbers=(((0,), (1,)), ((), ())),
        preferred_element_type=jnp.float32,
    )
    o_ref[...] = q + bq_ref[...]


def kernel(x, W_think, b_think, idx, W_q, b_q):
    batch, n_neurons = x.shape
    n_actions, n_motor = W_q.shape
    out_f, n_conn = idx.shape

    info = plsc.get_sparse_core_info()
    nw = info.num_cores * info.num_subcores
    npw = n_motor // nw
    rpw = npw * n_conn

    idx_m = idx[out_f - n_motor:]
    w_m = W_think[out_f - n_motor:]
    b_m = b_think[out_f - n_motor:]
    xt = x.T
    idx_w = idx_m.reshape(nw, rpw // 128, 128)
    w_b = jnp.broadcast_to(w_m.reshape(nw, rpw)[:, :, None], (nw, rpw, batch))
    b_b = jnp.broadcast_to(b_m.reshape(nw, npw)[:, :, None], (nw, npw, batch))

    motor_t = _sc_motor_kernel(n_motor, n_conn, batch, n_neurons)(
        idx_w, w_b, b_b, xt)

    q = pl.pallas_call(
        _q_head,
        out_shape=jax.ShapeDtypeStruct((batch, n_actions), jnp.float32),
    )(motor_t, W_q, b_q.reshape(1, n_actions))
    return q


# speedup vs baseline: 281.7939x; 1.5020x over previous
import functools

import jax
import jax.numpy as jnp
from jax import lax
from jax.experimental import pallas as pl
from jax.experimental.pallas import tpu as pltpu
from jax.experimental.pallas import tpu_sc as plsc

_CHUNK = 128


def _sc_motor_kernel(n_motor, n_conn, batch):
    info = plsc.get_sparse_core_info()
    nc, ns = info.num_cores, info.num_subcores
    nw = nc * ns
    npw = n_motor // nw
    rpw = npw * n_conn
    epw = rpw * batch
    n_chunks = epw // _CHUNK
    assert n_motor % nw == 0 and epw % _CHUNK == 0

    mesh = plsc.VectorSubcoreMesh(core_axis_name="c", subcore_axis_name="s")

    @functools.partial(
        pl.kernel,
        out_type=jax.ShapeDtypeStruct((n_motor, batch), jnp.float32),
        mesh=mesh,
        compiler_params=pltpu.CompilerParams(use_tc_tiling_on_sc=False),
        scratch_types=[
            pltpu.VMEM((n_chunks, _CHUNK), jnp.int32),
            pltpu.VMEM((epw,), jnp.float32),
            pltpu.VMEM((rpw, batch), jnp.float32),
            pltpu.VMEM((npw, batch), jnp.float32),
            pltpu.VMEM((npw, batch), jnp.float32),
            pltpu.SemaphoreType.DMA,
        ],
    )
    def k(idx_hbm, wb_hbm, bb_hbm, xf_hbm, out_hbm,
          idx_v, elems_v, w_v, b_v, out_v, sem):
        wid = lax.axis_index("s") * nc + lax.axis_index("c")
        pltpu.sync_copy(idx_hbm.at[wid], idx_v)
        pltpu.sync_copy(wb_hbm.at[wid], w_v)
        pltpu.sync_copy(bb_hbm.at[wid], b_v)

        def fire(c, carry):
            pltpu.async_copy(
                xf_hbm.at[idx_v.at[c]],
                elems_v.at[pl.ds(c * _CHUNK, _CHUNK)], sem)
            return carry

        lax.fori_loop(0, n_chunks, fire, 0, unroll=False)
        pltpu.make_async_copy(xf_hbm.at[pl.ds(0, epw)], elems_v, sem).wait()

        for n in range(npw):
            acc = b_v[n, :]
            for c in range(n_conn):
                r = n * n_conn + c
                acc = acc + elems_v[pl.ds(r * batch, batch)] * w_v[r, :]
            out_v[n, :] = 1.0 / (1.0 + jnp.exp(-acc))
        pltpu.sync_copy(out_v, out_hbm.at[pl.ds(wid * npw, npw)])

    return k


def _q_head(m_ref, wq_ref, bq_ref, o_ref):
    q = lax.dot_general(
        m_ref[...], wq_ref[...],
        dimension_numbers=(((0,), (1,)), ((), ())),
        preferred_element_type=jnp.float32,
    )
    o_ref[...] = q + bq_ref[...]


def kernel(x, W_think, b_think, idx, W_q, b_q):
    batch, n_neurons = x.shape
    n_actions, n_motor = W_q.shape
    out_f, n_conn = idx.shape

    info = plsc.get_sparse_core_info()
    nw = info.num_cores * info.num_subcores
    npw = n_motor // nw
    rpw = npw * n_conn

    idx_m = idx[out_f - n_motor:]
    w_m = W_think[out_f - n_motor:]
    b_m = b_think[out_f - n_motor:]
    xf = x.reshape(-1)
    offs = jnp.arange(batch, dtype=jnp.int32) * n_neurons
    idx16 = (idx_m.reshape(-1)[:, None] + offs[None, :]).reshape(
        nw, (rpw * batch) // _CHUNK, _CHUNK)
    w_b = jnp.broadcast_to(w_m.reshape(nw, rpw)[:, :, None], (nw, rpw, batch))
    b_b = jnp.broadcast_to(b_m.reshape(nw, npw)[:, :, None], (nw, npw, batch))

    motor_t = _sc_motor_kernel(n_motor, n_conn, batch)(idx16, w_b, b_b, xf)

    q = pl.pallas_call(
        _q_head,
        out_shape=jax.ShapeDtypeStruct((batch, n_actions), jnp.float32),
    )(motor_t, W_q, b_q.reshape(1, n_actions))
    return q


# speedup vs baseline: 325.5732x; 1.1554x over previous
import functools

import jax
import jax.numpy as jnp
from jax import lax
from jax.experimental import pallas as pl
from jax.experimental.pallas import tpu as pltpu
from jax.experimental.pallas import tpu_sc as plsc

_CHUNK = 128
_LANES = 16


def _splat(ref, p):
    g, lane = divmod(p, _LANES)
    v = ref[pl.ds(g * _LANES, _LANES)]
    return lax.gather(
        v, jnp.full((_LANES, 1), lane, jnp.int32),
        lax.GatherDimensionNumbers(
            offset_dims=(), collapsed_slice_dims=(0,), start_index_map=(0,)),
        slice_sizes=(1,), mode=lax.GatherScatterMode.PROMISE_IN_BOUNDS)


def _sc_motor_kernel(n_motor, n_conn, batch, n_neurons):
    info = plsc.get_sparse_core_info()
    nc, ns = info.num_cores, info.num_subcores
    nw = nc * ns
    npw = n_motor // nw
    rpw = npw * n_conn
    epw = rpw * batch
    n_chunks = epw // _CHUNK
    ppc = _CHUNK // batch
    assert n_motor % nw == 0 and epw % _CHUNK == 0 and batch == _LANES

    mesh = plsc.VectorSubcoreMesh(core_axis_name="c", subcore_axis_name="s")

    @functools.partial(
        pl.kernel,
        out_type=jax.ShapeDtypeStruct((n_motor, batch), jnp.float32),
        mesh=mesh,
        compiler_params=pltpu.CompilerParams(use_tc_tiling_on_sc=False),
        scratch_types=[
            pltpu.VMEM((rpw,), jnp.int32),
            pltpu.VMEM((rpw,), jnp.float32),
            pltpu.VMEM((_LANES,), jnp.float32),
            pltpu.VMEM((n_chunks, _CHUNK), jnp.int32),
            pltpu.VMEM((epw,), jnp.float32),
            pltpu.VMEM((npw, batch), jnp.float32),
            pltpu.SemaphoreType.DMA,
            pltpu.SemaphoreType.DMA,
        ],
    )
    def k(idx_hbm, w_hbm, b_hbm, xf_hbm, out_hbm,
          idx_l, w_l, b_l, idx_v, elems_v, out_v, sem, sem2):
        wid = lax.axis_index("s") * nc + lax.axis_index("c")
        pltpu.sync_copy(idx_hbm.at[pl.ds(wid * rpw, rpw)], idx_l)
        cp_w = pltpu.async_copy(w_hbm.at[pl.ds(wid * rpw, rpw)], w_l, sem2)
        cp_b = pltpu.async_copy(
            b_hbm.at[pl.ds(wid * npw, npw)], b_l.at[pl.ds(0, npw)], sem2)

        offs = lax.iota(jnp.int32, _LANES) * n_neurons
        for p in range(rpw):
            flat = _splat(idx_l, p) + offs
            idx_v[p // ppc, pl.ds((p % ppc) * batch, batch)] = flat

        def fire(c, carry):
            pltpu.async_copy(
                xf_hbm.at[idx_v.at[c]],
                elems_v.at[pl.ds(c * _CHUNK, _CHUNK)], sem)
            return carry

        lax.fori_loop(0, n_chunks, fire, 0, unroll=False)
        cp_w.wait()
        cp_b.wait()
        pltpu.make_async_copy(xf_hbm.at[pl.ds(0, epw)], elems_v, sem).wait()

        for n in range(npw):
            acc = _splat(b_l, n)
            for c in range(n_conn):
                p = n * n_conn + c
                acc = acc + elems_v[pl.ds(p * batch, batch)] * _splat(w_l, p)
            out_v[n, :] = 1.0 / (1.0 + jnp.exp(-acc))
        pltpu.sync_copy(out_v, out_hbm.at[pl.ds(wid * npw, npw)])

    return k


def _q_head(m_ref, wq_ref, bq_ref, o_ref):
    q = lax.dot_general(
        m_ref[...], wq_ref[...],
        dimension_numbers=(((0,), (1,)), ((), ())),
        preferred_element_type=jnp.float32,
    )
    o_ref[...] = q + bq_ref[...]


def kernel(x, W_think, b_think, idx, W_q, b_q):
    batch, n_neurons = x.shape
    n_actions, n_motor = W_q.shape
    out_f, n_conn = idx.shape

    idx_f = idx[out_f - n_motor:].reshape(-1)
    w_f = W_think[out_f - n_motor:].reshape(-1)
    b_m = b_think[out_f - n_motor:]
    xf = x.reshape(-1)

    motor_t = _sc_motor_kernel(n_motor, n_conn, batch, n_neurons)(
        idx_f, w_f, b_m, xf)

    q = pl.pallas_call(
        _q_head,
        out_shape=jax.ShapeDtypeStruct((batch, n_actions), jnp.float32),
    )(motor_t, W_q, b_q.reshape(1, n_actions))
    return q


# speedup vs baseline: 333.8260x; 1.0253x over previous
import functools

import jax
import jax.numpy as jnp
from jax import lax
from jax.experimental import pallas as pl
from jax.experimental.pallas import tpu as pltpu
from jax.experimental.pallas import tpu_sc as plsc

_CHUNK = 128
_LANES = 16


def _vsplat(v, lane):
    return lax.gather(
        v, jnp.full((_LANES, 1), lane, jnp.int32),
        lax.GatherDimensionNumbers(
            offset_dims=(), collapsed_slice_dims=(0,), start_index_map=(0,)),
        slice_sizes=(1,), mode=lax.GatherScatterMode.PROMISE_IN_BOUNDS)


def _sc_motor_kernel(n_motor, n_conn, batch, n_neurons):
    info = plsc.get_sparse_core_info()
    nc, ns = info.num_cores, info.num_subcores
    nw = nc * ns
    npw = n_motor // nw
    rpw = npw * n_conn
    epw = rpw * batch
    n_chunks = epw // _CHUNK
    ppc = _CHUNK // batch
    cpr = n_conn // ppc
    assert n_motor % nw == 0 and epw % _CHUNK == 0 and batch == _LANES
    assert n_conn % _LANES == 0 and n_motor % n_conn == 0

    mesh = plsc.VectorSubcoreMesh(core_axis_name="c", subcore_axis_name="s")

    @functools.partial(
        pl.kernel,
        out_type=jax.ShapeDtypeStruct((n_motor, batch), jnp.float32),
        mesh=mesh,
        compiler_params=pltpu.CompilerParams(use_tc_tiling_on_sc=False),
        scratch_types=[
            pltpu.VMEM((npw, n_conn), jnp.int32),
            pltpu.VMEM((npw, n_conn), jnp.int32),
            pltpu.VMEM((_LANES,), jnp.int32),
            pltpu.VMEM((n_chunks, _CHUNK), jnp.int32),
            pltpu.VMEM((epw,), jnp.float32),
            pltpu.VMEM((npw, batch), jnp.float32),
            pltpu.SemaphoreType.DMA,
            pltpu.SemaphoreType.DMA,
        ],
    )
    def k(packed_hbm, xf_hbm, out_hbm,
          idx_l, w_l, b_l, idx_v, elems_v, out_v, sem, sem2):
        wid = lax.axis_index("s") * nc + lax.axis_index("c")
        pltpu.sync_copy(packed_hbm.at[pl.ds(wid * npw, npw)], idx_l)
        cp_w = pltpu.async_copy(
            packed_hbm.at[pl.ds(n_motor + wid * npw, npw)], w_l, sem2)
        cp_b = pltpu.async_copy(
            packed_hbm.at[2 * n_motor + (wid * npw) // n_conn,
                          pl.ds((wid * npw) % n_conn, npw)],
            b_l.at[pl.ds(0, npw)], sem2)

        offs = lax.iota(jnp.int32, _LANES) * n_neurons
        for c in range(n_chunks):
            n = c // cpr
            col = (c % cpr) * ppc
            iv = idx_l[n, pl.ds((col // _LANES) * _LANES, _LANES)]
            for j in range(ppc):
                flat = _vsplat(iv, col % _LANES + j) + offs
                idx_v[c, pl.ds(j * batch, batch)] = flat
            pltpu.async_copy(
                xf_hbm.at[idx_v.at[c]],
                elems_v.at[pl.ds(c * _CHUNK, _CHUNK)], sem)

        cp_w.wait()
        cp_b.wait()
        pltpu.make_async_copy(xf_hbm.at[pl.ds(0, epw)], elems_v, sem).wait()

        bv = lax.bitcast_convert_type(b_l[...], jnp.float32)
        for n in range(npw):
            acc = _vsplat(bv, n)
            for h in range(n_conn // _LANES):
                wv = lax.bitcast_convert_type(
                    w_l[n, pl.ds(h * _LANES, _LANES)], jnp.float32)
                for j in range(_LANES):
                    p = n * n_conn + h * _LANES + j
                    acc = acc + (elems_v[pl.ds(p * batch, batch)]
                                 * _vsplat(wv, j))
            out_v[n, :] = 1.0 / (1.0 + jnp.exp(-acc))
        pltpu.sync_copy(out_v, out_hbm.at[pl.ds(wid * npw, npw)])

    return k


def _q_head(m_ref, wq_ref, bq_ref, o_ref):
    q = lax.dot_general(
        m_ref[...], wq_ref[...],
        dimension_numbers=(((0,), (1,)), ((), ())),
        preferred_element_type=jnp.float32,
    )
    o_ref[...] = q + bq_ref[...][None, :]


def kernel(x, W_think, b_think, idx, W_q, b_q):
    batch, n_neurons = x.shape
    n_actions, n_motor = W_q.shape
    out_f, n_conn = idx.shape

    lo = out_f - n_motor
    packed = jnp.concatenate([
        idx[lo:],
        lax.bitcast_convert_type(W_think[lo:], jnp.int32),
        lax.bitcast_convert_type(
            b_think[lo:].reshape(n_motor // n_conn, n_conn), jnp.int32),
    ], axis=0)
    xf = x.reshape(-1)

    motor_t = _sc_motor_kernel(n_motor, n_conn, batch, n_neurons)(packed, xf)

    q = pl.pallas_call(
        _q_head,
        out_shape=jax.ShapeDtypeStruct((batch, n_actions), jnp.float32),
    )(motor_t, W_q, b_q)
    return q


# speedup vs baseline: 335.7406x; 1.0057x over previous
import functools

import jax
import jax.numpy as jnp
from jax import lax
from jax.experimental import pallas as pl
from jax.experimental.pallas import tpu as pltpu
from jax.experimental.pallas import tpu_sc as plsc

_CHUNK = 128
_LANES = 16


def _vsplat(v, lane):
    return lax.gather(
        v, jnp.full((_LANES, 1), lane, jnp.int32),
        lax.GatherDimensionNumbers(
            offset_dims=(), collapsed_slice_dims=(0,), start_index_map=(0,)),
        slice_sizes=(1,), mode=lax.GatherScatterMode.PROMISE_IN_BOUNDS)


def _sc_motor_kernel(n_motor, n_conn, batch, n_neurons, b_lo):
    info = plsc.get_sparse_core_info()
    nc, ns = info.num_cores, info.num_subcores
    nw = nc * ns
    npw = n_motor // nw
    rpw = npw * n_conn
    epw = rpw * batch
    n_chunks = epw // _CHUNK
    ppc = _CHUNK // batch
    cpr = n_conn // ppc
    assert n_motor % nw == 0 and epw % _CHUNK == 0 and batch == _LANES
    assert n_conn % _LANES == 0 and npw <= _LANES

    mesh = plsc.VectorSubcoreMesh(core_axis_name="c", subcore_axis_name="s")

    @functools.partial(
        pl.kernel,
        out_type=jax.ShapeDtypeStruct((n_motor, batch), jnp.float32),
        mesh=mesh,
        compiler_params=pltpu.CompilerParams(use_tc_tiling_on_sc=False),
        scratch_types=[
            pltpu.VMEM((npw, n_conn), jnp.int32),
            pltpu.VMEM((npw, n_conn), jnp.float32),
            pltpu.VMEM((_LANES,), jnp.float32),
            pltpu.VMEM((n_chunks, _CHUNK), jnp.int32),
            pltpu.VMEM((epw,), jnp.float32),
            pltpu.VMEM((npw, batch), jnp.float32),
            pltpu.SemaphoreType.DMA,
            pltpu.SemaphoreType.DMA,
        ],
    )
    def k(idx_hbm, w_hbm, b_hbm, xf_hbm, out_hbm,
          idx_l, w_l, b_l, idx_v, elems_v, out_v, sem, sem2):
        wid = lax.axis_index("s") * nc + lax.axis_index("c")
        pltpu.sync_copy(idx_hbm.at[pl.ds(wid * npw, npw)], idx_l)
        cp_w = pltpu.async_copy(w_hbm.at[pl.ds(wid * npw, npw)], w_l, sem2)
        cp_b = pltpu.async_copy(
            b_hbm.at[pl.ds(b_lo + wid * npw, npw)],
            b_l.at[pl.ds(0, npw)], sem2)

        offs = lax.iota(jnp.int32, _LANES) * n_neurons
        for c in range(n_chunks):
            n = c // cpr
            col = (c % cpr) * ppc
            iv = idx_l[n, pl.ds((col // _LANES) * _LANES, _LANES)]
            for j in range(ppc):
                flat = _vsplat(iv, col % _LANES + j) + offs
                idx_v[c, pl.ds(j * batch, batch)] = flat
            pltpu.async_copy(
                xf_hbm.at[idx_v.at[c]],
                elems_v.at[pl.ds(c * _CHUNK, _CHUNK)], sem)

        cp_w.wait()
        cp_b.wait()
        pltpu.make_async_copy(xf_hbm.at[pl.ds(0, epw)], elems_v, sem).wait()

        bv = b_l[...]
        for n in range(npw):
            acc = _vsplat(bv, n)
            for h in range(n_conn // _LANES):
                wv = w_l[n, pl.ds(h * _LANES, _LANES)]
                for j in range(_LANES):
                    p = n * n_conn + h * _LANES + j
                    acc = acc + (elems_v[pl.ds(p * batch, batch)]
                                 * _vsplat(wv, j))
            out_v[n, :] = 1.0 / (1.0 + jnp.exp(-acc))
        pltpu.sync_copy(out_v, out_hbm.at[pl.ds(wid * npw, npw)])

    return k


def _q_head(m_ref, wq_ref, bq_ref, o_ref):
    q = lax.dot_general(
        m_ref[...], wq_ref[...],
        dimension_numbers=(((0,), (1,)), ((), ())),
        preferred_element_type=jnp.float32,
    )
    o_ref[...] = q + bq_ref[...][None, :]


def kernel(x, W_think, b_think, idx, W_q, b_q):
    batch, n_neurons = x.shape
    n_actions, n_motor = W_q.shape
    out_f, n_conn = idx.shape

    lo = out_f - n_motor
    idx_m = idx[lo:]
    w_m = W_think[lo:]
    xf = x.reshape(-1)

    motor_f = _sc_motor_kernel(n_motor, n_conn, batch, n_neurons, lo)(
        idx_m, w_m, b_think, xf)

    q = pl.pallas_call(
        _q_head,
        out_shape=jax.ShapeDtypeStruct((batch, n_actions), jnp.float32),
    )(motor_f, W_q, b_q)
    return q
